# baseline (speedup 1.0000x reference)
import functools

import jax
import jax.numpy as jnp
from jax import lax
from jax.experimental import pallas as pl
from jax.experimental.pallas import tpu as pltpu
from jax.experimental.pallas import tpu_sc as plsc

B, N, VOCAB, EMB = 16, 1024, 4096, 192
EMBP = 256

_NC, _NS = 2, 16
_NW = _NC * _NS
_CHUNK = 128


def _argmax_body(x_ref, dict_ref, tok_ref, dict_p_ref):
    xb = x_ref[0]
    m = jnp.max(xb, axis=-1, keepdims=True)
    iota = lax.broadcasted_iota(jnp.int32, xb.shape, 1)
    idx = jnp.min(jnp.where(xb == m, iota, VOCAB), axis=-1)
    tok_ref[...] = idx.astype(jnp.int32).reshape(N // 128, 128)

    @pl.when(pl.program_id(0) == 0)
    def _():
        dict_p_ref[:, :EMB] = dict_ref[...]
        dict_p_ref[:, EMB:] = jnp.zeros((VOCAB, EMBP - EMB), jnp.float32)


def _argmax_tokens(x, dictionary):
    return pl.pallas_call(
        _argmax_body,
        grid=(B,),
        in_specs=[
            pl.BlockSpec((1, N, VOCAB), lambda b: (b, 0, 0)),
            pl.BlockSpec((VOCAB, EMB), lambda b: (0, 0)),
        ],
        out_specs=[
            pl.BlockSpec((N // 128, 128), lambda b: (b, 0)),
            pl.BlockSpec((VOCAB, EMBP), lambda b: (0, 0)),
        ],
        out_shape=[
            jax.ShapeDtypeStruct((B * N // 128, 128), jnp.int32),
            jax.ShapeDtypeStruct((VOCAB, EMBP), jnp.float32),
        ],
    )(x, dictionary)


_BPW = B * N // _NW
_WPB = N // _BPW
_NGATH = _BPW // _CHUNK
_NSLOT = 3


def _make_sc_gather():
    mesh = plsc.VectorSubcoreMesh(core_axis_name="c", subcore_axis_name="s")

    @functools.partial(
        pl.kernel,
        mesh=mesh,
        out_type=jax.ShapeDtypeStruct((B, N, EMBP), jnp.float32),
        scratch_types=[
            pltpu.VMEM((_NGATH, _CHUNK), jnp.int32),
            pltpu.VMEM((_NSLOT, _CHUNK, EMBP), jnp.float32),
            pltpu.SemaphoreType.DMA,
            pltpu.SemaphoreType.DMA,
        ],
        compiler_params=pltpu.CompilerParams(use_tc_tiling_on_sc=True),
    )
    def sc_gather(table_hbm, idx_hbm, out_hbm, idx_v, rows_v, gsem, wsem):
        wid = lax.axis_index("s") * _NC + lax.axis_index("c")
        b = wid // _WPB
        noff = (wid % _WPB) * _BPW
        pltpu.sync_copy(idx_hbm.at[pl.ds(wid * _NGATH, _NGATH)], idx_v)
        gathers = []
        writes = []
        for j in range(_NGATH):
            s = j % _NSLOT
            if j >= _NSLOT:
                writes[j - _NSLOT].wait()
            gathers.append(
                pltpu.async_copy(table_hbm.at[idx_v.at[j]], rows_v.at[s], gsem)
            )
            jd = j - _NSLOT + 1
            if jd >= 0:
                gathers[jd].wait()
                writes.append(
                    pltpu.async_copy(
                        rows_v.at[jd % _NSLOT],
                        out_hbm.at[b, pl.ds(noff + jd * _CHUNK, _CHUNK)],
                        wsem,
                    )
                )
        for jd in range(_NGATH - _NSLOT + 1, _NGATH):
            gathers[jd].wait()
            writes.append(
                pltpu.async_copy(
                    rows_v.at[jd % _NSLOT],
                    out_hbm.at[b, pl.ds(noff + jd * _CHUNK, _CHUNK)],
                    wsem,
                )
            )
        for w in writes[max(0, _NGATH - _NSLOT):]:
            w.wait()

    return sc_gather


_SC_GATHER_CACHE = []


def kernel(x, dictionary):
    if not _SC_GATHER_CACHE:
        _SC_GATHER_CACHE.append(_make_sc_gather())
    tokens, dict_p = _argmax_tokens(x, dictionary)
    out_p = _SC_GATHER_CACHE[0](dict_p, tokens)
    return out_p[:, :, :EMB]


# speedup vs baseline: 1.0199x; 1.0199x over previous
import functools

import jax
import jax.numpy as jnp
from jax import lax
from jax.experimental import pallas as pl
from jax.experimental.pallas import tpu as pltpu
from jax.experimental.pallas import tpu_sc as plsc

B, N, VOCAB, EMB = 16, 1024, 4096, 192
EMBP = 256

_NC, _NS = 2, 16
_NW = _NC * _NS
_CHUNK = 128


def _argmax_body(x_ref, tok_ref):
    xb = x_ref[0]
    m = jnp.max(xb, axis=-1, keepdims=True)
    iota = lax.broadcasted_iota(jnp.int32, xb.shape, 1)
    idx = jnp.min(jnp.where(xb == m, iota, VOCAB), axis=-1)
    tok_ref[...] = idx.astype(jnp.int32).reshape(N // 128, 128)


def _argmax_tokens(x):
    return pl.pallas_call(
        _argmax_body,
        grid=(B,),
        in_specs=[pl.BlockSpec((1, N, VOCAB), lambda b: (b, 0, 0))],
        out_specs=pl.BlockSpec((N // 128, 128), lambda b: (b, 0)),
        out_shape=jax.ShapeDtypeStruct((B * N // 128, 128), jnp.int32),
    )(x)


_BPW = B * N // _NW
_WPB = N // _BPW
_NGATH = _BPW // _CHUNK
_NSLOT = 3


def _make_sc_gather():
    mesh = plsc.VectorSubcoreMesh(core_axis_name="c", subcore_axis_name="s")

    @functools.partial(
        pl.kernel,
        mesh=mesh,
        out_type=jax.ShapeDtypeStruct((B, N, EMBP), jnp.float32),
        scratch_types=[
            pltpu.VMEM((_NGATH, _CHUNK), jnp.int32),
            pltpu.VMEM((_NSLOT, _CHUNK, EMBP), jnp.float32),
            pltpu.SemaphoreType.DMA,
            pltpu.SemaphoreType.DMA,
        ],
        compiler_params=pltpu.CompilerParams(use_tc_tiling_on_sc=True),
    )
    def sc_gather(table_hbm, idx_hbm, out_hbm, idx_v, rows_v, gsem, wsem):
        wid = lax.axis_index("s") * _NC + lax.axis_index("c")
        b = wid // _WPB
        noff = (wid % _WPB) * _BPW
        pltpu.sync_copy(idx_hbm.at[pl.ds(wid * _NGATH, _NGATH)], idx_v)
        gathers = []
        writes = []
        for j in range(_NGATH):
            s = j % _NSLOT
            if j >= _NSLOT:
                writes[j - _NSLOT].wait()
            gathers.append(
                pltpu.async_copy(table_hbm.at[idx_v.at[j]], rows_v.at[s], gsem)
            )
            jd = j - _NSLOT + 1
            if jd >= 0:
                gathers[jd].wait()
                writes.append(
                    pltpu.async_copy(
                        rows_v.at[jd % _NSLOT],
                        out_hbm.at[b, pl.ds(noff + jd * _CHUNK, _CHUNK)],
                        wsem,
                    )
                )
        for jd in range(_NGATH - _NSLOT + 1, _NGATH):
            gathers[jd].wait()
            writes.append(
                pltpu.async_copy(
                    rows_v.at[jd % _NSLOT],
                    out_hbm.at[b, pl.ds(noff + jd * _CHUNK, _CHUNK)],
                    wsem,
                )
            )
        for w in writes[max(0, _NGATH - _NSLOT):]:
            w.wait()

    return sc_gather


_SC_GATHER_CACHE = []


def kernel(x, dictionary):
    if not _SC_GATHER_CACHE:
        _SC_GATHER_CACHE.append(_make_sc_gather())
    tokens = _argmax_tokens(x)
    dict_p = jnp.pad(dictionary, ((0, 0), (0, EMBP - EMB)))
    out_p = _SC_GATHER_CACHE[0](dict_p, tokens)
    return out_p[:, :, :EMB]
